# Initial kernel scaffold; baseline (speedup 1.0000x reference)
import functools

import jax
import jax.numpy as jnp
from jax import lax
from jax.experimental import pallas as pl
from jax.experimental.pallas import tpu as pltpu
from jax.experimental.pallas import tpu_sc as plsc

N = 50000
E = 800000
NG = 2500
T = 10
IE = 32
H = 64
DEC = 128
OL = 25

NPAD = 50176
DEGN = 50048
JIDX = 50016
ROWS_PER_TILE = 392
EP = 16 * 50176
CONV_BLK = 49
DEG_PER_TILE = EP // 32
F32 = jnp.float32


def _lk(x):
    return jnp.maximum(x, 0.1 * x)


def _mesh():
    return plsc.VectorSubcoreMesh(core_axis_name="c", subcore_axis_name="s")


def _deg_body(dst1, zeros, out, hist, idx):
    c = lax.axis_index("c")
    s = lax.axis_index("s")
    w = s * 2 + c
    pltpu.sync_copy(zeros, hist)
    ones = jnp.ones((16,), F32)

    def blk(b, carry):
        base = w * DEG_PER_TILE + b * 512
        pltpu.sync_copy(dst1.at[pl.ds(base, 512)], idx)
        for g in range(32):
            ii = idx[pl.ds(g * 16, 16)]
            plsc.addupdate_scatter(hist, [ii], ones)
        return carry

    lax.fori_loop(0, 49, blk, 0)
    pltpu.sync_copy(hist, out.at[pl.ds(w * DEGN, DEGN)])


def _deg_sc(dstp, zeros_deg):
    k = pl.kernel(
        _deg_body,
        mesh=_mesh(),
        out_type=jax.ShapeDtypeStruct((32 * DEGN,), F32),
        scratch_types=[
            pltpu.VMEM((DEGN,), F32),
            pltpu.VMEM((512,), jnp.int32),
        ],
    )
    return k(dstp, zeros_deg)


def _conv_body(src2, dst2, zlo, zhi, zeros, out, sidx, didx, rows, accum, sem):
    c = lax.axis_index("c")
    s = lax.axis_index("s")
    slab = NPAD // 16
    pltpu.sync_copy(zeros.at[pl.ds(s * slab, slab)],
                    accum.at[pl.ds(s * slab, slab)])
    plsc.subcore_barrier()

    def run(ztab):
        def blk(b, carry):
            rowbase = s * ROWS_PER_TILE + b * 8
            pltpu.sync_copy(src2.at[pl.ds(rowbase, 8)], sidx)
            pltpu.sync_copy(dst2.at[pl.ds(rowbase, 8)], didx)
            cps = [
                pltpu.async_copy(ztab.at[sidx.at[j]],
                                 rows.at[pl.ds(j * 128, 128)], sem)
                for j in range(8)
            ]
            for cp in cps:
                cp.wait()
            for j in range(8):
                pltpu.sync_copy(rows.at[pl.ds(j * 128, 128)],
                                accum.at[didx.at[j]], add=True)
            return carry

        lax.fori_loop(0, CONV_BLK, blk, 0)

    @pl.when(c == 0)
    def _():
        run(zlo)

    @pl.when(c == 1)
    def _():
        run(zhi)

    plsc.subcore_barrier()
    pltpu.sync_copy(accum.at[pl.ds(s * slab, slab)],
                    out.at[pl.ds(c * NPAD + s * slab, slab)])


def _conv_sc(src2, dst2, zlo, zhi, zeros_conv):
    k = pl.kernel(
        _conv_body,
        mesh=_mesh(),
        out_type=jax.ShapeDtypeStruct((2 * NPAD, 32), F32),
        scratch_types=[
            pltpu.VMEM((8, 128), jnp.int32),
            pltpu.VMEM((8, 128), jnp.int32),
            pltpu.VMEM((1024, 32), F32),
            pltpu.VMEM_SHARED((NPAD, 32), F32),
            pltpu.SemaphoreType.DMA,
        ],
    )
    return k(src2, dst2, zlo, zhi, zeros_conv)


_NB = 2000
_GB = 640
_NGP = 2560


def _enc_body(x_ref, wip, bip, wihT, whhT, bih, bhh, wdyn, bdyn, hist_ref):
    xall = x_ref[...]
    B = xall.shape[0]
    Wip = wip[...]
    Bip = bip[...]
    WihT = wihT[...]
    WhhT = whhT[...]
    bi = bih[...]
    bh = bhh[...]
    h = jnp.zeros((B, H), F32)
    for t in range(T):
        xt = xall[:, 2 * t:2 * t + 2]
        emb = _lk(jnp.dot(xt, Wip, preferred_element_type=F32) + Bip)
        gi = jnp.dot(emb, WihT, preferred_element_type=F32) + bi
        gh = jnp.dot(h, WhhT, preferred_element_type=F32) + bh
        r = jax.nn.sigmoid(gi[:, :H] + gh[:, :H])
        z = jax.nn.sigmoid(gi[:, H:2 * H] + gh[:, H:2 * H])
        n = jnp.tanh(gi[:, 2 * H:] + r * gh[:, 2 * H:])
        h = (1.0 - z) * n + z * h
    hist_ref[...] = _lk(jnp.dot(_lk(h), wdyn[...],
                                preferred_element_type=F32) + bdyn[...])


def _full(shape):
    nd = len(shape)
    return pl.BlockSpec(shape, lambda i: (0,) * nd)


def _encoder(x2, Wip, bip, WihT, WhhT, bih, bhh, Wdyn, bdyn):
    grid = (N // _NB,)
    return pl.pallas_call(
        _enc_body,
        grid=grid,
        in_specs=[
            pl.BlockSpec((_NB, 2 * T), lambda i: (i, 0)),
            _full(Wip.shape), _full(bip.shape), _full(WihT.shape),
            _full(WhhT.shape), _full(bih.shape), _full(bhh.shape),
            _full(Wdyn.shape), _full(bdyn.shape),
        ],
        out_specs=pl.BlockSpec((_NB, H), lambda i: (i, 0)),
        out_shape=jax.ShapeDtypeStruct((N, H), F32),
    )(x2, Wip, bip, WihT, WhhT, bih, bhh, Wdyn, bdyn)


def _mid1_body(degp, hist, wg1, dinv_ref, y1_ref, z1_ref):
    deg = jnp.sum(degp[...], axis=0) + 1.0
    dv = lax.rsqrt(deg)[:, None]
    y1 = jnp.dot(hist[...], wg1[...], preferred_element_type=F32)
    dinv_ref[...] = dv
    y1_ref[...] = y1
    z1_ref[...] = dv * y1


def _mid1(degp, hist, Wg1):
    grid = (N // _NB,)
    return pl.pallas_call(
        _mid1_body,
        grid=grid,
        in_specs=[
            pl.BlockSpec((32, _NB), lambda i: (0, i)),
            pl.BlockSpec((_NB, H), lambda i: (i, 0)),
            _full(Wg1.shape),
        ],
        out_specs=[
            pl.BlockSpec((_NB, 1), lambda i: (i, 0)),
            pl.BlockSpec((_NB, H), lambda i: (i, 0)),
            pl.BlockSpec((_NB, H), lambda i: (i, 0)),
        ],
        out_shape=[
            jax.ShapeDtypeStruct((N, 1), F32),
            jax.ShapeDtypeStruct((N, H), F32),
            jax.ShapeDtypeStruct((N, H), F32),
        ],
    )(degp, hist, Wg1)


def _mid2_body(a1, dinv, y1, hist, bg1, wg2, y2_ref, z2_ref):
    dv = dinv[...]
    g1 = dv * a1[...] + dv * dv * y1[...] + bg1[...]
    W = wg2[...]
    y2 = (jnp.dot(g1, W[:H], preferred_element_type=F32)
          + jnp.dot(hist[...], W[H:], preferred_element_type=F32))
    y2_ref[...] = y2
    z2_ref[...] = dv * y2


def _mid2(a1, dinv, y1, hist, bg1, Wg2):
    grid = (N // _NB,)
    return pl.pallas_call(
        _mid2_body,
        grid=grid,
        in_specs=[
            pl.BlockSpec((_NB, H), lambda i: (i, 0)),
            pl.BlockSpec((_NB, 1), lambda i: (i, 0)),
            pl.BlockSpec((_NB, H), lambda i: (i, 0)),
            pl.BlockSpec((_NB, H), lambda i: (i, 0)),
            _full(bg1.shape),
            _full(Wg2.shape),
        ],
        out_specs=[
            pl.BlockSpec((_NB, H), lambda i: (i, 0)),
            pl.BlockSpec((_NB, H), lambda i: (i, 0)),
        ],
        out_shape=[
            jax.ShapeDtypeStruct((N, H), F32),
            jax.ShapeDtypeStruct((N, H), F32),
        ],
    )(a1, dinv, y1, hist, bg1, Wg2)


def _dec_body(a2, y2, dinv, hist, bg2, wnb, bnb, wih0T, whh0T, b0,
              wih1T, whh1T, b1, wop, bop, out_ref, h2seq):
    dv = dinv[...]
    g2 = dv * a2[...] + dv * dv * y2[...] + bg2[...]
    hs = hist[...]
    gat = _lk(jnp.dot(jnp.concatenate([g2, hs], 1), wnb[...],
                      preferred_element_type=F32) + bnb[...])
    enc = jnp.concatenate([hs, gat], 1)
    gx1 = jnp.dot(enc, wih0T[...], preferred_element_type=F32) + b0[...]
    W0 = whh0T[...]
    W1i = wih1T[...]
    W1h = whh1T[...]
    bb1 = b1[...]
    B = enc.shape[0]
    z = jnp.zeros((B, DEC), F32)

    def step(t, carry):
        h1, c1, h2, c2 = carry
        g = gx1 + jnp.dot(h1, W0, preferred_element_type=F32)
        i = jax.nn.sigmoid(g[:, :DEC])
        f = jax.nn.sigmoid(g[:, DEC:2 * DEC])
        gg = jnp.tanh(g[:, 2 * DEC:3 * DEC])
        o = jax.nn.sigmoid(g[:, 3 * DEC:])
        c1n = f * c1 + i * gg
        h1n = o * jnp.tanh(c1n)
        gq = (jnp.dot(h1n, W1i, preferred_element_type=F32)
              + jnp.dot(h2, W1h, preferred_element_type=F32) + bb1)
        i2 = jax.nn.sigmoid(gq[:, :DEC])
        f2 = jax.nn.sigmoid(gq[:, DEC:2 * DEC])
        gg2 = jnp.tanh(gq[:, 2 * DEC:3 * DEC])
        o2 = jax.nn.sigmoid(gq[:, 3 * DEC:])
        c2n = f2 * c2 + i2 * gg2
        h2n = o2 * jnp.tanh(c2n)
        h2seq[:, pl.ds(t, 1), :] = h2n[:, None, :]
        return (h1n, c1n, h2n, c2n)

    lax.fori_loop(0, OL, step, (z, z, z, z))
    hsq = h2seq[...].reshape(B * OL, DEC)
    out_ref[...] = (jnp.dot(hsq, wop[...], preferred_element_type=F32)
                    + bop[...]).reshape(B, OL, 2)


def _decoder(a2t, y2t, dinvt, histt, bg2, Wnb, bnb, Wih0T, Whh0T, b0,
             Wih1T, Whh1T, b1, Wop, bop):
    grid = (_NGP // _GB,)
    return pl.pallas_call(
        _dec_body,
        grid=grid,
        in_specs=[
            pl.BlockSpec((_GB, H), lambda i: (i, 0)),
            pl.BlockSpec((_GB, H), lambda i: (i, 0)),
            pl.BlockSpec((_GB, 1), lambda i: (i, 0)),
            pl.BlockSpec((_GB, H), lambda i: (i, 0)),
            _full(bg2.shape), _full(Wnb.shape), _full(bnb.shape),
            _full(Wih0T.shape), _full(Whh0T.shape), _full(b0.shape),
            _full(Wih1T.shape), _full(Whh1T.shape), _full(b1.shape),
            _full(Wop.shape), _full(bop.shape),
        ],
        out_specs=pl.BlockSpec((_GB, OL, 2), lambda i: (i, 0, 0)),
        out_shape=jax.ShapeDtypeStruct((_NGP, OL, 2), F32),
        scratch_shapes=[pltpu.VMEM((_GB, OL, DEC), F32)],
    )(a2t, y2t, dinvt, histt, bg2, Wnb, bnb, Wih0T, Whh0T, b0,
      Wih1T, Whh1T, b1, Wop, bop)


def kernel(x, edge_index, batch, W_ip, b_ip, W_ih_enc, W_hh_enc, b_ih_enc,
           b_hh_enc, W_dyn, b_dyn, W_g1, b_g1, W_g2, b_g2, W_nb, b_nb,
           W_ih0, W_hh0, b_ih0, b_hh0, W_ih1, W_hh1, b_ih1, b_hh1,
           W_op, b_op):
    x2 = x.reshape(N, 2 * T)
    src = edge_index[0].astype(jnp.int32)
    dst = edge_index[1].astype(jnp.int32)
    padlen = EP - E
    srcp = jnp.concatenate([src, jnp.full((padlen,), JIDX, jnp.int32)])
    dstp = jnp.concatenate([dst, jnp.full((padlen,), JIDX, jnp.int32)])
    src2 = srcp.reshape(EP // 128, 128)
    dst2 = dstp.reshape(EP // 128, 128)
    zeros_deg = jnp.zeros((DEGN,), F32)
    zeros_conv = jnp.zeros((NPAD, 32), F32)

    degp = _deg_sc(dstp, zeros_deg).reshape(32, DEGN)[:, :N]
    hist = _encoder(x2, W_ip, b_ip[None, :], W_ih_enc.T, W_hh_enc.T,
                    b_ih_enc[None, :], b_hh_enc[None, :], W_dyn,
                    b_dyn[None, :])

    dinv, y1, z1 = _mid1(degp, hist, W_g1)

    z1p = jnp.pad(z1, ((0, NPAD - N), (0, 0)))
    a1f = _conv_sc(src2, dst2, z1p[:, :32], z1p[:, 32:], zeros_conv)
    a1 = jnp.concatenate([a1f[:N], a1f[NPAD:NPAD + N]], axis=1)

    y2, z2 = _mid2(a1, dinv, y1, hist, b_g1[None, :], W_g2)

    z2p = jnp.pad(z2, ((0, NPAD - N), (0, 0)))
    a2f = _conv_sc(src2, dst2, z2p[:, :32], z2p[:, 32:], zeros_conv)
    a2 = jnp.concatenate([a2f[:N], a2f[NPAD:NPAD + N]], axis=1)

    a2t = a2.reshape(NG, N // NG, H)[:, 0]
    y2t = y2.reshape(NG, N // NG, H)[:, 0]
    histt = hist.reshape(NG, N // NG, H)[:, 0]
    dinvt = dinv.reshape(NG, N // NG, 1)[:, 0]

    gp = _NGP - NG
    a2t = jnp.pad(a2t, ((0, gp), (0, 0)))
    y2t = jnp.pad(y2t, ((0, gp), (0, 0)))
    histt = jnp.pad(histt, ((0, gp), (0, 0)))
    dinvt = jnp.pad(dinvt, ((0, gp), (0, 0)))

    out = _decoder(a2t, y2t, dinvt, histt, b_g2[None, :], W_nb,
                   b_nb[None, :], W_ih0.T, W_hh0.T,
                   (b_ih0 + b_hh0)[None, :], W_ih1.T, W_hh1.T,
                   (b_ih1 + b_hh1)[None, :], W_op, b_op[None, :])
    return out[:NG]


# speedup vs baseline: 14.6293x; 14.6293x over previous
import functools

import jax
import jax.numpy as jnp
from jax import lax
from jax.experimental import pallas as pl
from jax.experimental.pallas import tpu as pltpu
from jax.experimental.pallas import tpu_sc as plsc

N = 50000
E = 800000
NG = 2500
T = 10
IE = 32
H = 64
DEC = 128
OL = 25

NPAD = 50176
DEGN = 50048
JIDX = 50016
ROWS_PER_TILE = 392
EP = 16 * 50176
CONV_BLK = 98
DEG_PER_TILE = EP // 32
F32 = jnp.float32


def _lk(x):
    return jnp.maximum(x, 0.1 * x)


def _mesh():
    return plsc.VectorSubcoreMesh(core_axis_name="c", subcore_axis_name="s")


def _deg_body(dst1, zeros, out, hist, idx):
    c = lax.axis_index("c")
    s = lax.axis_index("s")
    w = s * 2 + c
    pltpu.sync_copy(zeros, hist)
    ones = jnp.ones((16,), F32)

    def blk(b, carry):
        base = w * DEG_PER_TILE + b * 512
        pltpu.sync_copy(dst1.at[pl.ds(base, 512)], idx)
        for g in range(32):
            ii = idx[pl.ds(g * 16, 16)]
            plsc.addupdate_scatter(hist, [ii], ones)
        return carry

    lax.fori_loop(0, 49, blk, 0)
    pltpu.sync_copy(hist, out.at[pl.ds(w * DEGN, DEGN)])


def _deg_sc(dstp, zeros_deg):
    k = pl.kernel(
        _deg_body,
        mesh=_mesh(),
        out_type=jax.ShapeDtypeStruct((32 * DEGN,), F32),
        scratch_types=[
            pltpu.VMEM((DEGN,), F32),
            pltpu.VMEM((512,), jnp.int32),
        ],
        compiler_params=pltpu.CompilerParams(needs_layout_passes=False),
    )
    return k(dstp, zeros_deg)


def _conv_body(src2, dst2, zlo, zhi, zeros, out, sidx, didx, rows, accum, sem):
    c = lax.axis_index("c")
    s = lax.axis_index("s")
    slab = NPAD // 16
    pltpu.sync_copy(zeros.at[pl.ds(s * slab, slab)],
                    accum.at[pl.ds(s * slab, slab)])
    plsc.subcore_barrier()

    def run(ztab):
        def blk(b, carry):
            rowbase = s * ROWS_PER_TILE + b * 4
            pltpu.sync_copy(src2.at[pl.ds(rowbase, 4)], sidx)
            pltpu.sync_copy(dst2.at[pl.ds(rowbase, 4)], didx)
            cps = [
                pltpu.async_copy(ztab.at[sidx.at[j]],
                                 rows.at[pl.ds(j * 128, 128)], sem)
                for j in range(4)
            ]
            for cp in cps:
                cp.wait()
            for j in range(4):
                pltpu.sync_copy(rows.at[pl.ds(j * 128, 128)],
                                accum.at[didx.at[j]], add=True)
            return carry

        lax.fori_loop(0, CONV_BLK, blk, 0)

    @pl.when(c == 0)
    def _():
        run(zlo)

    @pl.when(c == 1)
    def _():
        run(zhi)

    plsc.subcore_barrier()
    pltpu.sync_copy(accum.at[pl.ds(s * slab, slab)],
                    out.at[pl.ds(c * NPAD + s * slab, slab)])


def _conv_sc(src2, dst2, zlo, zhi, zeros_conv):
    k = pl.kernel(
        _conv_body,
        mesh=_mesh(),
        out_type=jax.ShapeDtypeStruct((2 * NPAD, 32), F32),
        scratch_types=[
            pltpu.VMEM((4, 128), jnp.int32),
            pltpu.VMEM((4, 128), jnp.int32),
            pltpu.VMEM((512, 32), F32),
            pltpu.VMEM_SHARED((NPAD, 32), F32),
            pltpu.SemaphoreType.DMA,
        ],
        compiler_params=pltpu.CompilerParams(
            needs_layout_passes=False, use_tc_tiling_on_sc=False),
    )
    return k(src2, dst2, zlo, zhi, zeros_conv)


_NB = 2000
_GB = 640
_NGP = 2560


def _enc_body(x_ref, wip, bip, wihT, whhT, bih, bhh, wdyn, bdyn, hist_ref):
    xall = x_ref[...]
    B = xall.shape[0]
    Wip = wip[...]
    Bip = bip[...]
    WihT = wihT[...]
    WhhT = whhT[...]
    bi = bih[...]
    bh = bhh[...]
    h = jnp.zeros((B, H), F32)
    for t in range(T):
        xt = xall[:, 2 * t:2 * t + 2]
        emb = _lk(jnp.dot(xt, Wip, preferred_element_type=F32) + Bip)
        gi = jnp.dot(emb, WihT, preferred_element_type=F32) + bi
        gh = jnp.dot(h, WhhT, preferred_element_type=F32) + bh
        r = jax.nn.sigmoid(gi[:, :H] + gh[:, :H])
        z = jax.nn.sigmoid(gi[:, H:2 * H] + gh[:, H:2 * H])
        n = jnp.tanh(gi[:, 2 * H:] + r * gh[:, 2 * H:])
        h = (1.0 - z) * n + z * h
    hist_ref[...] = _lk(jnp.dot(_lk(h), wdyn[...],
                                preferred_element_type=F32) + bdyn[...])


def _full(shape):
    nd = len(shape)
    return pl.BlockSpec(shape, lambda i: (0,) * nd)


def _encoder(x2, Wip, bip, WihT, WhhT, bih, bhh, Wdyn, bdyn):
    grid = (N // _NB,)
    return pl.pallas_call(
        _enc_body,
        grid=grid,
        in_specs=[
            pl.BlockSpec((_NB, 2 * T), lambda i: (i, 0)),
            _full(Wip.shape), _full(bip.shape), _full(WihT.shape),
            _full(WhhT.shape), _full(bih.shape), _full(bhh.shape),
            _full(Wdyn.shape), _full(bdyn.shape),
        ],
        out_specs=pl.BlockSpec((_NB, H), lambda i: (i, 0)),
        out_shape=jax.ShapeDtypeStruct((N, H), F32),
    )(x2, Wip, bip, WihT, WhhT, bih, bhh, Wdyn, bdyn)


def _mid1_body(degp, hist, wg1, dinv_ref, y1_ref, z1_ref):
    deg = jnp.sum(degp[...], axis=1) + 1.0
    dv = lax.rsqrt(deg)[:, None]
    y1 = jnp.dot(hist[...], wg1[...], preferred_element_type=F32)
    dinv_ref[...] = dv
    y1_ref[...] = y1
    z1_ref[...] = dv * y1


def _mid1(degp, hist, Wg1):
    grid = (N // _NB,)
    return pl.pallas_call(
        _mid1_body,
        grid=grid,
        in_specs=[
            pl.BlockSpec((_NB, 32), lambda i: (i, 0)),
            pl.BlockSpec((_NB, H), lambda i: (i, 0)),
            _full(Wg1.shape),
        ],
        out_specs=[
            pl.BlockSpec((_NB, 1), lambda i: (i, 0)),
            pl.BlockSpec((_NB, H), lambda i: (i, 0)),
            pl.BlockSpec((_NB, H), lambda i: (i, 0)),
        ],
        out_shape=[
            jax.ShapeDtypeStruct((N, 1), F32),
            jax.ShapeDtypeStruct((N, H), F32),
            jax.ShapeDtypeStruct((N, H), F32),
        ],
    )(degp, hist, Wg1)


def _mid2_body(a1, dinv, y1, hist, bg1, wg2, y2_ref, z2_ref):
    dv = dinv[...]
    g1 = dv * a1[...] + dv * dv * y1[...] + bg1[...]
    W = wg2[...]
    y2 = (jnp.dot(g1, W[:H], preferred_element_type=F32)
          + jnp.dot(hist[...], W[H:], preferred_element_type=F32))
    y2_ref[...] = y2
    z2_ref[...] = dv * y2


def _mid2(a1, dinv, y1, hist, bg1, Wg2):
    grid = (N // _NB,)
    return pl.pallas_call(
        _mid2_body,
        grid=grid,
        in_specs=[
            pl.BlockSpec((_NB, H), lambda i: (i, 0)),
            pl.BlockSpec((_NB, 1), lambda i: (i, 0)),
            pl.BlockSpec((_NB, H), lambda i: (i, 0)),
            pl.BlockSpec((_NB, H), lambda i: (i, 0)),
            _full(bg1.shape),
            _full(Wg2.shape),
        ],
        out_specs=[
            pl.BlockSpec((_NB, H), lambda i: (i, 0)),
            pl.BlockSpec((_NB, H), lambda i: (i, 0)),
        ],
        out_shape=[
            jax.ShapeDtypeStruct((N, H), F32),
            jax.ShapeDtypeStruct((N, H), F32),
        ],
    )(a1, dinv, y1, hist, bg1, Wg2)


def _dec_body(a2, y2, dinv, hist, bg2, wnb, bnb, wih0T, whh0T, b0,
              wih1T, whh1T, b1, wop, bop, out_ref, h2seq):
    dv = dinv[...]
    g2 = dv * a2[...] + dv * dv * y2[...] + bg2[...]
    hs = hist[...]
    gat = _lk(jnp.dot(jnp.concatenate([g2, hs], 1), wnb[...],
                      preferred_element_type=F32) + bnb[...])
    enc = jnp.concatenate([hs, gat], 1)
    gx1 = jnp.dot(enc, wih0T[...], preferred_element_type=F32) + b0[...]
    W0 = whh0T[...]
    W1i = wih1T[...]
    W1h = whh1T[...]
    bb1 = b1[...]
    B = enc.shape[0]
    z = jnp.zeros((B, DEC), F32)

    def step(t, carry):
        h1, c1, h2, c2 = carry
        g = gx1 + jnp.dot(h1, W0, preferred_element_type=F32)
        i = jax.nn.sigmoid(g[:, :DEC])
        f = jax.nn.sigmoid(g[:, DEC:2 * DEC])
        gg = jnp.tanh(g[:, 2 * DEC:3 * DEC])
        o = jax.nn.sigmoid(g[:, 3 * DEC:])
        c1n = f * c1 + i * gg
        h1n = o * jnp.tanh(c1n)
        gq = (jnp.dot(h1n, W1i, preferred_element_type=F32)
              + jnp.dot(h2, W1h, preferred_element_type=F32) + bb1)
        i2 = jax.nn.sigmoid(gq[:, :DEC])
        f2 = jax.nn.sigmoid(gq[:, DEC:2 * DEC])
        gg2 = jnp.tanh(gq[:, 2 * DEC:3 * DEC])
        o2 = jax.nn.sigmoid(gq[:, 3 * DEC:])
        c2n = f2 * c2 + i2 * gg2
        h2n = o2 * jnp.tanh(c2n)
        h2seq[:, pl.ds(t, 1), :] = h2n[:, None, :]
        return (h1n, c1n, h2n, c2n)

    lax.fori_loop(0, OL, step, (z, z, z, z))
    hsq = h2seq[...].reshape(B * OL, DEC)
    out_ref[...] = (jnp.dot(hsq, wop[...], preferred_element_type=F32)
                    + bop[...]).reshape(B, OL, 2)


def _decoder(a2t, y2t, dinvt, histt, bg2, Wnb, bnb, Wih0T, Whh0T, b0,
             Wih1T, Whh1T, b1, Wop, bop):
    grid = (_NGP // _GB,)
    return pl.pallas_call(
        _dec_body,
        grid=grid,
        in_specs=[
            pl.BlockSpec((_GB, H), lambda i: (i, 0)),
            pl.BlockSpec((_GB, H), lambda i: (i, 0)),
            pl.BlockSpec((_GB, 1), lambda i: (i, 0)),
            pl.BlockSpec((_GB, H), lambda i: (i, 0)),
            _full(bg2.shape), _full(Wnb.shape), _full(bnb.shape),
            _full(Wih0T.shape), _full(Whh0T.shape), _full(b0.shape),
            _full(Wih1T.shape), _full(Whh1T.shape), _full(b1.shape),
            _full(Wop.shape), _full(bop.shape),
        ],
        out_specs=pl.BlockSpec((_GB, OL, 2), lambda i: (i, 0, 0)),
        out_shape=jax.ShapeDtypeStruct((_NGP, OL, 2), F32),
        scratch_shapes=[pltpu.VMEM((_GB, OL, DEC), F32)],
    )(a2t, y2t, dinvt, histt, bg2, Wnb, bnb, Wih0T, Whh0T, b0,
      Wih1T, Whh1T, b1, Wop, bop)


def kernel(x, edge_index, batch, W_ip, b_ip, W_ih_enc, W_hh_enc, b_ih_enc,
           b_hh_enc, W_dyn, b_dyn, W_g1, b_g1, W_g2, b_g2, W_nb, b_nb,
           W_ih0, W_hh0, b_ih0, b_hh0, W_ih1, W_hh1, b_ih1, b_hh1,
           W_op, b_op):
    x2 = x.reshape(N, 2 * T)
    src = edge_index[0].astype(jnp.int32)
    dst = edge_index[1].astype(jnp.int32)
    padlen = EP - E
    srcp = jnp.concatenate([src, jnp.full((padlen,), JIDX, jnp.int32)])
    dstp = jnp.concatenate([dst, jnp.full((padlen,), JIDX, jnp.int32)])
    src2 = srcp.reshape(EP // 128, 128)
    dst2 = dstp.reshape(EP // 128, 128)
    zeros_deg = jnp.zeros((DEGN,), F32)
    zeros_conv = jnp.zeros((NPAD, 32), F32)

    degp = _deg_sc(dstp, zeros_deg).reshape(32, DEGN)[:, :N].T
    hist = _encoder(x2, W_ip, b_ip[None, :], W_ih_enc.T, W_hh_enc.T,
                    b_ih_enc[None, :], b_hh_enc[None, :], W_dyn,
                    b_dyn[None, :])

    dinv, y1, z1 = _mid1(degp, hist, W_g1)

    z1p = jnp.pad(z1, ((0, NPAD - N), (0, 0)))
    a1f = _conv_sc(src2, dst2, z1p[:, :32], z1p[:, 32:], zeros_conv)
    a1 = jnp.concatenate([a1f[:N], a1f[NPAD:NPAD + N]], axis=1)

    y2, z2 = _mid2(a1, dinv, y1, hist, b_g1[None, :], W_g2)

    z2p = jnp.pad(z2, ((0, NPAD - N), (0, 0)))
    a2f = _conv_sc(src2, dst2, z2p[:, :32], z2p[:, 32:], zeros_conv)
    a2 = jnp.concatenate([a2f[:N], a2f[NPAD:NPAD + N]], axis=1)

    a2t = a2.reshape(NG, N // NG, H)[:, 0]
    y2t = y2.reshape(NG, N // NG, H)[:, 0]
    histt = hist.reshape(NG, N // NG, H)[:, 0]
    dinvt = dinv.reshape(NG, N // NG, 1)[:, 0]

    gp = _NGP - NG
    a2t = jnp.pad(a2t, ((0, gp), (0, 0)))
    y2t = jnp.pad(y2t, ((0, gp), (0, 0)))
    histt = jnp.pad(histt, ((0, gp), (0, 0)))
    dinvt = jnp.pad(dinvt, ((0, gp), (0, 0)))

    out = _decoder(a2t, y2t, dinvt, histt, b_g2[None, :], W_nb,
                   b_nb[None, :], W_ih0.T, W_hh0.T,
                   (b_ih0 + b_hh0)[None, :], W_ih1.T, W_hh1.T,
                   (b_ih1 + b_hh1)[None, :], W_op, b_op[None, :])
    return out[:NG]
